# baseline (speedup 1.0000x reference)
import jax
import jax.numpy as jnp
from jax.experimental import pallas as pl
from jax.experimental.pallas import tpu as pltpu


def _pick_bm(n: int, target: int) -> int:
    for bm in range(target, 7, -1):
        if n % bm == 0 and bm % 8 == 0:
            return bm
    return n


def _xw_kernel(x_ref, w_ref, out_ref):
    out_ref[...] = jnp.dot(
        x_ref[...].astype(jnp.bfloat16),
        w_ref[...].astype(jnp.bfloat16),
        preferred_element_type=jnp.float32,
    )


def _layer1_kernel(adj_ref, s1_ref, b1_ref, w2_ref, out_ref):
    acc = jnp.dot(
        adj_ref[...].astype(jnp.bfloat16),
        s1_ref[...].astype(jnp.bfloat16),
        preferred_element_type=jnp.float32,
    )
    h = jnp.maximum(acc + b1_ref[...], 0.0)
    out_ref[...] = jnp.dot(
        h.astype(jnp.bfloat16),
        w2_ref[...].astype(jnp.bfloat16),
        preferred_element_type=jnp.float32,
    )


def _layer2_kernel(adj_ref, s2_ref, b2_ref, out_ref):
    acc = jnp.dot(
        adj_ref[...].astype(jnp.bfloat16),
        s2_ref[...].astype(jnp.bfloat16),
        preferred_element_type=jnp.float32,
    )
    out_ref[...] = acc + b2_ref[...]


def kernel(x, adj, W1, b1, W2, b2):
    n, din = x.shape
    dh = W1.shape[1]
    de = W2.shape[1]

    b1r = b1.reshape(1, dh)
    b2r = b2.reshape(1, de)

    bm1 = _pick_bm(n, 2000)
    s1 = pl.pallas_call(
        _xw_kernel,
        grid=(n // bm1,),
        in_specs=[
            pl.BlockSpec((bm1, din), lambda i: (i, 0)),
            pl.BlockSpec((din, dh), lambda i: (0, 0)),
        ],
        out_specs=pl.BlockSpec((bm1, dh), lambda i: (i, 0)),
        out_shape=jax.ShapeDtypeStruct((n, dh), jnp.float32),
    )(x, W1)

    bm = _pick_bm(n, 500)
    grid = (n // bm,)
    s2 = pl.pallas_call(
        _layer1_kernel,
        grid=grid,
        in_specs=[
            pl.BlockSpec((bm, n), lambda i: (i, 0)),
            pl.BlockSpec((n, dh), lambda i: (0, 0)),
            pl.BlockSpec((1, dh), lambda i: (0, 0)),
            pl.BlockSpec((dh, de), lambda i: (0, 0)),
        ],
        out_specs=pl.BlockSpec((bm, de), lambda i: (i, 0)),
        out_shape=jax.ShapeDtypeStruct((n, de), jnp.float32),
        compiler_params=pltpu.CompilerParams(
            dimension_semantics=("arbitrary",),
        ),
    )(adj, s1, b1r, W2)

    out = pl.pallas_call(
        _layer2_kernel,
        grid=grid,
        in_specs=[
            pl.BlockSpec((bm, n), lambda i: (i, 0)),
            pl.BlockSpec((n, de), lambda i: (0, 0)),
            pl.BlockSpec((1, de), lambda i: (0, 0)),
        ],
        out_specs=pl.BlockSpec((bm, de), lambda i: (i, 0)),
        out_shape=jax.ShapeDtypeStruct((n, de), jnp.float32),
        compiler_params=pltpu.CompilerParams(
            dimension_semantics=("arbitrary",),
        ),
    )(adj, s2, b2r)

    return out
